# baseline (speedup 1.0000x reference)
import jax
import jax.numpy as jnp
from jax import lax
from jax.experimental import pallas as pl
from jax.experimental.pallas import tpu as pltpu
from jax.experimental.pallas import tpu_sc as plsc

_BS, _N, _D = 16, 16384, 128

_NB_TC = 15
_NB_SC = _BS - _NB_TC

_NSUB = 32
_SPB = _NSUB // _NB_SC
_LH = _N // _SPB
_W = 128
_NWIN = _LH // _W

_BLK = 16384
_NB = _N // _BLK


def _sc_body(pred_hbm, target_hbm, mask_hbm, out_hbm,
             mask_v, idx_v, rows_p0, rows_t0, rows_p1, rows_t1, acc_v,
             sem_p0, sem_t0, sem_p1, sem_t1):
    c = lax.axis_index("c")
    s = lax.axis_index("s")
    wid = s * 2 + c
    b = _NB_TC + wid // _SPB
    h = wid % _SPB
    start = b * _N + h * _LH

    pltpu.sync_copy(mask_hbm.at[pl.ds(start, _LH)], mask_v)

    safe = jnp.full((16,), start, jnp.int32)

    def prefill(i, carry):
        idx_v[pl.ds(i * 16, 16)] = safe
        return carry

    lax.fori_loop(0, _LH // 16, prefill, 0)

    def compact(i, off):
        mi = mask_v[pl.ds(i * 16, 16)]
        keep = mi != 0
        one = keep.astype(jnp.int32)
        csum = plsc.cumsum(one)
        pos = csum + (off - 1)
        rowids = lax.iota(jnp.int32, 16) + (start + i * 16)
        plsc.store_scatter(idx_v, [pos], rowids, mask=keep)
        return off + jnp.sum(one)

    count = lax.fori_loop(0, _LH // 16, compact, jnp.int32(0))

    nwin = (count + _W - 1) // _W

    zero = jnp.zeros((16,), jnp.float32)
    acc_v[pl.ds(0, 16)] = zero
    acc_v[pl.ds(16, 16)] = zero
    acc_v[pl.ds(32, 16)] = zero

    slots = ((rows_p0, rows_t0, sem_p0, sem_t0),
             (rows_p1, rows_t1, sem_p1, sem_t1))

    def issue(w, slot):
        rp, rt, sp, st = slot
        idx_slice = idx_v.at[pl.ds(w * _W, _W)]
        pltpu.async_copy(pred_hbm.at[idx_slice], rp, sp)
        pltpu.async_copy(target_hbm.at[idx_slice], rt, st)

    @pl.when(nwin > 0)
    def _():
        issue(0, slots[0])

    for w in range(_NWIN):
        rp, rt, sp, st = slots[w % 2]

        if w + 1 < _NWIN:
            @pl.when(w + 1 < nwin)
            def _(w=w):
                issue(w + 1, slots[(w + 1) % 2])

        @pl.when(w < nwin)
        def _(w=w, rp=rp, rt=rt, sp=sp, st=st):
            idx_slice = idx_v.at[pl.ds(w * _W, _W)]
            pltpu.make_async_copy(pred_hbm.at[idx_slice], rp, sp).wait()
            pltpu.make_async_copy(target_hbm.at[idx_slice], rt, st).wait()
            nrows = jnp.minimum(count - w * _W, _W)

            def row(r, a):
                a1, a2, a3 = a
                for ch in range(_D // 16):
                    pch = rp[r, pl.ds(ch * 16, 16)]
                    tch = rt[r, pl.ds(ch * 16, 16)]
                    a1 = a1 + pch * tch
                    a2 = a2 + pch * pch
                    a3 = a3 + tch * tch
                return (a1, a2, a3)

            a0 = (acc_v[pl.ds(0, 16)], acc_v[pl.ds(16, 16)], acc_v[pl.ds(32, 16)])
            a1, a2, a3 = lax.fori_loop(0, nrows, row, a0)
            acc_v[pl.ds(0, 16)] = a1
            acc_v[pl.ds(16, 16)] = a2
            acc_v[pl.ds(32, 16)] = a3

    acc_v[pl.ds(48, 16)] = jnp.full((16,), 1.0, jnp.float32) * count.astype(jnp.float32)
    pltpu.sync_copy(acc_v, out_hbm.at[wid])


def _sc_call(pred_flat, target_flat, mask_flat):
    mesh = plsc.VectorSubcoreMesh(core_axis_name="c", subcore_axis_name="s")
    kern = pl.kernel(
        _sc_body,
        out_type=jax.ShapeDtypeStruct((_NSUB, 64), jnp.float32),
        mesh=mesh,
        scratch_types=[
            pltpu.VMEM((_LH,), jnp.int32),
            pltpu.VMEM((_LH,), jnp.int32),
            pltpu.VMEM((_W, _D), jnp.float32),
            pltpu.VMEM((_W, _D), jnp.float32),
            pltpu.VMEM((_W, _D), jnp.float32),
            pltpu.VMEM((_W, _D), jnp.float32),
            pltpu.VMEM((64,), jnp.float32),
            pltpu.SemaphoreType.DMA,
            pltpu.SemaphoreType.DMA,
            pltpu.SemaphoreType.DMA,
            pltpu.SemaphoreType.DMA,
        ],
        compiler_params=pltpu.CompilerParams(needs_layout_passes=False),
    )
    return kern(pred_flat, target_flat, mask_flat)


def _tc_body(mask_ref, pred_ref, target_ref, out_ref, acc_ref, cnt_ref):
    b = pl.program_id(0)
    i = pl.program_id(1)

    @pl.when(jnp.logical_and(b == 0, i == 0))
    def _():
        out_ref[0, 0] = 0.0

    @pl.when(i == 0)
    def _():
        acc_ref[...] = jnp.zeros_like(acc_ref)
        cnt_ref[0] = 0.0

    m = (mask_ref[0, 0, :] != 0).astype(jnp.float32)
    mf = m[:, None]
    p = pred_ref[0]
    t = target_ref[0]
    mp = (p * mf).reshape(_BLK // 8, 8, _D)
    mt = (t * mf).reshape(_BLK // 8, 8, _D)
    pr = p.reshape(_BLK // 8, 8, _D)
    tr = t.reshape(_BLK // 8, 8, _D)
    acc_ref[0] += jnp.sum(mp * tr, axis=0)
    acc_ref[1] += jnp.sum(mp * pr, axis=0)
    acc_ref[2] += jnp.sum(mt * tr, axis=0)
    cnt_ref[0] += jnp.sum(m)

    @pl.when(i == _NB - 1)
    def _():
        dot = jnp.sum(acc_ref[0])
        pp = jnp.sum(acc_ref[1])
        tt = jnp.sum(acc_ref[2])
        cnt = cnt_ref[0]
        denom = jnp.sqrt(pp) * jnp.sqrt(tt)
        safe = jnp.where(denom > 0.0, denom, 1.0)
        loss = jnp.where(cnt > 0.0, -dot / safe, 0.0)
        out_ref[0, 0] += loss / _BS


def _tc_call(mask3, pred, target):
    return pl.pallas_call(
        _tc_body,
        grid=(_NB_TC, _NB),
        in_specs=[
            pl.BlockSpec((1, 1, _BLK), lambda b, i: (b * _NB + i, 0, 0)),
            pl.BlockSpec((1, _BLK, _D), lambda b, i: (b, i, 0)),
            pl.BlockSpec((1, _BLK, _D), lambda b, i: (b, i, 0)),
        ],
        out_specs=pl.BlockSpec(memory_space=pltpu.SMEM),
        out_shape=jax.ShapeDtypeStruct((1, 1), jnp.float32),
        scratch_shapes=[pltpu.VMEM((3, 8, _D), jnp.float32),
                        pltpu.SMEM((1,), jnp.float32)],
    )(mask3, pred, target)


def kernel(pred, target, mask):
    pred_flat = pred.reshape(_BS * _N, _D)
    target_flat = target.reshape(_BS * _N, _D)
    mask_flat = mask.reshape(_BS * _N)
    mask3 = mask.reshape(_BS * _NB, 1, _BLK)

    sc = _sc_call(pred_flat, target_flat, mask_flat)
    tc = _tc_call(mask3, pred, target)

    scr = sc.reshape(_NB_SC, _SPB, 4, 16)
    dot = jnp.sum(scr[:, :, 0, :], axis=(1, 2))
    pp = jnp.sum(scr[:, :, 1, :], axis=(1, 2))
    tt = jnp.sum(scr[:, :, 2, :], axis=(1, 2))
    cnt = jnp.sum(scr[:, :, 3, 0], axis=1)

    denom = jnp.sqrt(pp) * jnp.sqrt(tt)
    safe = jnp.where(denom > 0.0, denom, 1.0)
    losses = jnp.where(cnt > 0.0, -dot / safe, 0.0)
    return tc[0, 0] + jnp.sum(losses) / _BS


# speedup vs baseline: 1.0144x; 1.0144x over previous
import jax
import jax.numpy as jnp
from jax import lax
from jax.experimental import pallas as pl
from jax.experimental.pallas import tpu as pltpu
from jax.experimental.pallas import tpu_sc as plsc

_BS, _N, _D = 16, 16384, 128

_NB_TC = 14
_NB_SC = _BS - _NB_TC

_NSUB = 32
_SPB = _NSUB // _NB_SC
_LH = _N // _SPB
_W = 128
_NWIN = _LH // _W

_BLK = 16384
_NB = _N // _BLK


def _sc_body(pred_hbm, target_hbm, mask_hbm, out_hbm,
             mask_v, idx_v, rows_p0, rows_t0, rows_p1, rows_t1, acc_v,
             sem_p0, sem_t0, sem_p1, sem_t1):
    c = lax.axis_index("c")
    s = lax.axis_index("s")
    wid = s * 2 + c
    b = _NB_TC + wid // _SPB
    h = wid % _SPB
    start = b * _N + h * _LH

    pltpu.sync_copy(mask_hbm.at[pl.ds(start, _LH)], mask_v)

    safe = jnp.full((16,), start, jnp.int32)

    def prefill(i, carry):
        idx_v[pl.ds(i * 16, 16)] = safe
        return carry

    lax.fori_loop(0, _LH // 16, prefill, 0)

    def compact(i, off):
        mi = mask_v[pl.ds(i * 16, 16)]
        keep = mi != 0
        one = keep.astype(jnp.int32)
        csum = plsc.cumsum(one)
        pos = csum + (off - 1)
        rowids = lax.iota(jnp.int32, 16) + (start + i * 16)
        plsc.store_scatter(idx_v, [pos], rowids, mask=keep)
        return off + jnp.sum(one)

    count = lax.fori_loop(0, _LH // 16, compact, jnp.int32(0))

    nwin = (count + _W - 1) // _W

    zero = jnp.zeros((16,), jnp.float32)
    acc_v[pl.ds(0, 16)] = zero
    acc_v[pl.ds(16, 16)] = zero
    acc_v[pl.ds(32, 16)] = zero

    slots = ((rows_p0, rows_t0, sem_p0, sem_t0),
             (rows_p1, rows_t1, sem_p1, sem_t1))

    def issue(w, slot):
        rp, rt, sp, st = slot
        idx_slice = idx_v.at[pl.ds(w * _W, _W)]
        pltpu.async_copy(pred_hbm.at[idx_slice], rp, sp)
        pltpu.async_copy(target_hbm.at[idx_slice], rt, st)

    @pl.when(nwin > 0)
    def _():
        issue(0, slots[0])

    for w in range(_NWIN):
        rp, rt, sp, st = slots[w % 2]

        if w + 1 < _NWIN:
            @pl.when(w + 1 < nwin)
            def _(w=w):
                issue(w + 1, slots[(w + 1) % 2])

        @pl.when(w < nwin)
        def _(w=w, rp=rp, rt=rt, sp=sp, st=st):
            idx_slice = idx_v.at[pl.ds(w * _W, _W)]
            pltpu.make_async_copy(pred_hbm.at[idx_slice], rp, sp).wait()
            pltpu.make_async_copy(target_hbm.at[idx_slice], rt, st).wait()
            nrows = jnp.minimum(count - w * _W, _W)

            def row(r, a):
                a1, a2, a3 = a
                for ch in range(_D // 16):
                    pch = rp[r, pl.ds(ch * 16, 16)]
                    tch = rt[r, pl.ds(ch * 16, 16)]
                    a1 = a1 + pch * tch
                    a2 = a2 + pch * pch
                    a3 = a3 + tch * tch
                return (a1, a2, a3)

            a0 = (acc_v[pl.ds(0, 16)], acc_v[pl.ds(16, 16)], acc_v[pl.ds(32, 16)])
            a1, a2, a3 = lax.fori_loop(0, nrows, row, a0)
            acc_v[pl.ds(0, 16)] = a1
            acc_v[pl.ds(16, 16)] = a2
            acc_v[pl.ds(32, 16)] = a3

    acc_v[pl.ds(48, 16)] = jnp.full((16,), 1.0, jnp.float32) * count.astype(jnp.float32)
    pltpu.sync_copy(acc_v, out_hbm.at[wid])


def _sc_call(pred_flat, target_flat, mask_flat):
    mesh = plsc.VectorSubcoreMesh(core_axis_name="c", subcore_axis_name="s")
    kern = pl.kernel(
        _sc_body,
        out_type=jax.ShapeDtypeStruct((_NSUB, 64), jnp.float32),
        mesh=mesh,
        scratch_types=[
            pltpu.VMEM((_LH,), jnp.int32),
            pltpu.VMEM((_LH,), jnp.int32),
            pltpu.VMEM((_W, _D), jnp.float32),
            pltpu.VMEM((_W, _D), jnp.float32),
            pltpu.VMEM((_W, _D), jnp.float32),
            pltpu.VMEM((_W, _D), jnp.float32),
            pltpu.VMEM((64,), jnp.float32),
            pltpu.SemaphoreType.DMA,
            pltpu.SemaphoreType.DMA,
            pltpu.SemaphoreType.DMA,
            pltpu.SemaphoreType.DMA,
        ],
        compiler_params=pltpu.CompilerParams(needs_layout_passes=False),
    )
    return kern(pred_flat, target_flat, mask_flat)


def _tc_body(mask_ref, pred_ref, target_ref, out_ref, acc_ref, cnt_ref):
    b = pl.program_id(0)
    i = pl.program_id(1)

    @pl.when(jnp.logical_and(b == 0, i == 0))
    def _():
        out_ref[0, 0] = 0.0

    @pl.when(i == 0)
    def _():
        acc_ref[...] = jnp.zeros_like(acc_ref)
        cnt_ref[0] = 0.0

    m = (mask_ref[0, 0, :] != 0).astype(jnp.float32)
    mf = m[:, None]
    p = pred_ref[0]
    t = target_ref[0]
    mp = (p * mf).reshape(_BLK // 8, 8, _D)
    mt = (t * mf).reshape(_BLK // 8, 8, _D)
    pr = p.reshape(_BLK // 8, 8, _D)
    tr = t.reshape(_BLK // 8, 8, _D)
    acc_ref[0] += jnp.sum(mp * tr, axis=0)
    acc_ref[1] += jnp.sum(mp * pr, axis=0)
    acc_ref[2] += jnp.sum(mt * tr, axis=0)
    cnt_ref[0] += jnp.sum(m)

    @pl.when(i == _NB - 1)
    def _():
        dot = jnp.sum(acc_ref[0])
        pp = jnp.sum(acc_ref[1])
        tt = jnp.sum(acc_ref[2])
        cnt = cnt_ref[0]
        denom = jnp.sqrt(pp) * jnp.sqrt(tt)
        safe = jnp.where(denom > 0.0, denom, 1.0)
        loss = jnp.where(cnt > 0.0, -dot / safe, 0.0)
        out_ref[0, 0] += loss / _BS


def _tc_call(mask3, pred, target, b0, nb):
    return pl.pallas_call(
        _tc_body,
        grid=(nb, _NB),
        in_specs=[
            pl.BlockSpec((1, 1, _BLK), lambda b, i: ((b0 + b) * _NB + i, 0, 0)),
            pl.BlockSpec((1, _BLK, _D), lambda b, i: (b0 + b, i, 0)),
            pl.BlockSpec((1, _BLK, _D), lambda b, i: (b0 + b, i, 0)),
        ],
        out_specs=pl.BlockSpec(memory_space=pltpu.SMEM),
        out_shape=jax.ShapeDtypeStruct((1, 1), jnp.float32),
        scratch_shapes=[pltpu.VMEM((3, 8, _D), jnp.float32),
                        pltpu.SMEM((1,), jnp.float32)],
    )(mask3, pred, target)


def kernel(pred, target, mask):
    pred_flat = pred.reshape(_BS * _N, _D)
    target_flat = target.reshape(_BS * _N, _D)
    mask_flat = mask.reshape(_BS * _N)
    mask3 = mask.reshape(_BS * _NB, 1, _BLK)

    sc = _sc_call(pred_flat, target_flat, mask_flat)
    half = _NB_TC // 2
    tc1 = _tc_call(mask3, pred, target, 0, half)
    tc2 = _tc_call(mask3, pred, target, half, _NB_TC - half)

    scr = sc.reshape(_NB_SC, _SPB, 4, 16)
    dot = jnp.sum(scr[:, :, 0, :], axis=(1, 2))
    pp = jnp.sum(scr[:, :, 1, :], axis=(1, 2))
    tt = jnp.sum(scr[:, :, 2, :], axis=(1, 2))
    cnt = jnp.sum(scr[:, :, 3, 0], axis=1)

    denom = jnp.sqrt(pp) * jnp.sqrt(tt)
    safe = jnp.where(denom > 0.0, denom, 1.0)
    losses = jnp.where(cnt > 0.0, -dot / safe, 0.0)
    return tc1[0, 0] + tc2[0, 0] + jnp.sum(losses) / _BS


# speedup vs baseline: 1.0615x; 1.0464x over previous
import jax
import jax.numpy as jnp
from jax import lax
from jax.experimental import pallas as pl
from jax.experimental.pallas import tpu as pltpu
from jax.experimental.pallas import tpu_sc as plsc

_BS, _N, _D = 16, 16384, 128

_NB_TC = 14
_NB_SC = _BS - _NB_TC

_NSUB = 32
_SPB = _NSUB // _NB_SC
_LH = _N // _SPB
_W = 128
_NWIN = _LH // _W

_BLK = 16384
_NB = _N // _BLK


def _sc_body(pred_hbm, target_hbm, mask_hbm, out_hbm,
             mask_v, idx_v, rows_p0, rows_t0, rows_p1, rows_t1, acc_v,
             sem_p0, sem_t0, sem_p1, sem_t1):
    c = lax.axis_index("c")
    s = lax.axis_index("s")
    wid = s * 2 + c
    b = _NB_TC + wid // _SPB
    h = wid % _SPB
    start = b * _N + h * _LH

    pltpu.sync_copy(mask_hbm.at[pl.ds(start, _LH)], mask_v)

    safe = jnp.full((16,), start, jnp.int32)

    def prefill(i, carry):
        idx_v[pl.ds(i * 16, 16)] = safe
        return carry

    lax.fori_loop(0, _LH // 16, prefill, 0)

    def compact(i, off):
        mi = mask_v[pl.ds(i * 16, 16)]
        keep = mi != 0
        one = keep.astype(jnp.int32)
        csum = plsc.cumsum(one)
        pos = csum + (off - 1)
        rowids = lax.iota(jnp.int32, 16) + (start + i * 16)
        plsc.store_scatter(idx_v, [pos], rowids, mask=keep)
        return off + jnp.sum(one)

    count = lax.fori_loop(0, _LH // 16, compact, jnp.int32(0))

    nwin = (count + _W - 1) // _W

    zero = jnp.zeros((16,), jnp.float32)
    acc_v[pl.ds(0, 16)] = zero
    acc_v[pl.ds(16, 16)] = zero
    acc_v[pl.ds(32, 16)] = zero

    slots = ((rows_p0, rows_t0, sem_p0, sem_t0),
             (rows_p1, rows_t1, sem_p1, sem_t1))

    def issue(w, slot):
        rp, rt, sp, st = slot
        idx_slice = idx_v.at[pl.ds(w * _W, _W)]
        pltpu.async_copy(pred_hbm.at[idx_slice], rp, sp)
        pltpu.async_copy(target_hbm.at[idx_slice], rt, st)

    @pl.when(nwin > 0)
    def _():
        issue(0, slots[0])

    for w in range(_NWIN):
        rp, rt, sp, st = slots[w % 2]

        if w + 1 < _NWIN:
            @pl.when(w + 1 < nwin)
            def _(w=w):
                issue(w + 1, slots[(w + 1) % 2])

        @pl.when(w < nwin)
        def _(w=w, rp=rp, rt=rt, sp=sp, st=st):
            idx_slice = idx_v.at[pl.ds(w * _W, _W)]
            pltpu.make_async_copy(pred_hbm.at[idx_slice], rp, sp).wait()
            pltpu.make_async_copy(target_hbm.at[idx_slice], rt, st).wait()
            nrows = jnp.minimum(count - w * _W, _W)

            def row(r, a):
                a1, a2, a3 = a
                for ch in range(_D // 16):
                    pch = rp[r, pl.ds(ch * 16, 16)]
                    tch = rt[r, pl.ds(ch * 16, 16)]
                    a1 = a1 + pch * tch
                    a2 = a2 + pch * pch
                    a3 = a3 + tch * tch
                return (a1, a2, a3)

            a0 = (acc_v[pl.ds(0, 16)], acc_v[pl.ds(16, 16)], acc_v[pl.ds(32, 16)])
            a1, a2, a3 = lax.fori_loop(0, nrows, row, a0)
            acc_v[pl.ds(0, 16)] = a1
            acc_v[pl.ds(16, 16)] = a2
            acc_v[pl.ds(32, 16)] = a3

    acc_v[pl.ds(48, 16)] = jnp.full((16,), 1.0, jnp.float32) * count.astype(jnp.float32)
    pltpu.sync_copy(acc_v, out_hbm.at[wid])


def _sc_call(pred_flat, target_flat, mask_flat):
    mesh = plsc.VectorSubcoreMesh(core_axis_name="c", subcore_axis_name="s")
    kern = pl.kernel(
        _sc_body,
        out_type=jax.ShapeDtypeStruct((_NSUB, 64), jnp.float32),
        mesh=mesh,
        scratch_types=[
            pltpu.VMEM((_LH,), jnp.int32),
            pltpu.VMEM((_LH,), jnp.int32),
            pltpu.VMEM((_W, _D), jnp.float32),
            pltpu.VMEM((_W, _D), jnp.float32),
            pltpu.VMEM((_W, _D), jnp.float32),
            pltpu.VMEM((_W, _D), jnp.float32),
            pltpu.VMEM((64,), jnp.float32),
            pltpu.SemaphoreType.DMA,
            pltpu.SemaphoreType.DMA,
            pltpu.SemaphoreType.DMA,
            pltpu.SemaphoreType.DMA,
        ],
        compiler_params=pltpu.CompilerParams(needs_layout_passes=False),
    )
    return kern(pred_flat, target_flat, mask_flat)


def _tc_body(mask_ref, pred_ref, target_ref, out_ref, acc_ref, cnt_ref):
    b = pl.program_id(0)
    i = pl.program_id(1)

    @pl.when(jnp.logical_and(b == 0, i == 0))
    def _():
        out_ref[0, 0] = 0.0

    @pl.when(i == 0)
    def _():
        acc_ref[...] = jnp.zeros_like(acc_ref)
        cnt_ref[0] = 0.0

    m = (mask_ref[0, 0, :] != 0).astype(jnp.float32)
    mf = m[:, None]
    p = pred_ref[0]
    t = target_ref[0]
    mp = (p * mf).reshape(_BLK // 8, 8, _D)
    mt = (t * mf).reshape(_BLK // 8, 8, _D)
    pr = p.reshape(_BLK // 8, 8, _D)
    tr = t.reshape(_BLK // 8, 8, _D)
    acc_ref[0] += jnp.sum(mp * tr, axis=0)
    acc_ref[1] += jnp.sum(mp * pr, axis=0)
    acc_ref[2] += jnp.sum(mt * tr, axis=0)
    cnt_ref[0] += jnp.sum(m)

    @pl.when(i == _NB - 1)
    def _():
        dot = jnp.sum(acc_ref[0])
        pp = jnp.sum(acc_ref[1])
        tt = jnp.sum(acc_ref[2])
        cnt = cnt_ref[0]
        denom = jnp.sqrt(pp) * jnp.sqrt(tt)
        safe = jnp.where(denom > 0.0, denom, 1.0)
        loss = jnp.where(cnt > 0.0, -dot / safe, 0.0)
        out_ref[0, 0] += loss / _BS


def _tc_call(mask3, pred, target, b0, nb):
    return pl.pallas_call(
        _tc_body,
        grid=(nb, _NB),
        in_specs=[
            pl.BlockSpec((1, 1, _BLK), lambda b, i: ((b0 + b) * _NB + i, 0, 0)),
            pl.BlockSpec((1, _BLK, _D), lambda b, i: (b0 + b, i, 0)),
            pl.BlockSpec((1, _BLK, _D), lambda b, i: (b0 + b, i, 0)),
        ],
        out_specs=pl.BlockSpec(memory_space=pltpu.SMEM),
        out_shape=jax.ShapeDtypeStruct((1, 1), jnp.float32),
        scratch_shapes=[pltpu.VMEM((3, 8, _D), jnp.float32),
                        pltpu.SMEM((1,), jnp.float32)],
    )(mask3, pred, target)


def kernel(pred, target, mask):
    pred_flat = pred.reshape(_BS * _N, _D)
    target_flat = target.reshape(_BS * _N, _D)
    mask_flat = mask.reshape(_BS * _N)
    mask3 = mask.reshape(_BS * _NB, 1, _BLK)

    sc = _sc_call(pred_flat, target_flat, mask_flat)
    tc = _tc_call(mask3, pred, target, 0, _NB_TC)

    scr = sc.reshape(_NB_SC, _SPB, 4, 16)
    dot = jnp.sum(scr[:, :, 0, :], axis=(1, 2))
    pp = jnp.sum(scr[:, :, 1, :], axis=(1, 2))
    tt = jnp.sum(scr[:, :, 2, :], axis=(1, 2))
    cnt = jnp.sum(scr[:, :, 3, 0], axis=1)

    denom = jnp.sqrt(pp) * jnp.sqrt(tt)
    safe = jnp.where(denom > 0.0, denom, 1.0)
    losses = jnp.where(cnt > 0.0, -dot / safe, 0.0)
    return tc[0, 0] + jnp.sum(losses) / _BS


# speedup vs baseline: 1.0647x; 1.0030x over previous
import jax
import jax.numpy as jnp
from jax import lax
from jax.experimental import pallas as pl
from jax.experimental.pallas import tpu as pltpu
from jax.experimental.pallas import tpu_sc as plsc

_BS, _N, _D = 16, 16384, 128

_NB_TC = 14
_NB_SC = _BS - _NB_TC

_NSUB = 32
_SPB = _NSUB // _NB_SC
_LH = _N // _SPB
_W = 128
_NWIN = _LH // _W

_BLK = 16384
_NB = _N // _BLK


def _sc_body(pred_hbm, target_hbm, mask_hbm, out_hbm,
             mask_v, idx_v, rows_p0, rows_t0, rows_p1, rows_t1, acc_v,
             sem_p0, sem_t0, sem_p1, sem_t1):
    c = lax.axis_index("c")
    s = lax.axis_index("s")
    wid = s * 2 + c
    b = _NB_TC + wid // _SPB
    h = wid % _SPB
    start = b * _N + h * _LH

    pltpu.sync_copy(mask_hbm.at[pl.ds(start, _LH)], mask_v)

    safe = jnp.full((16,), start, jnp.int32)

    def prefill(i, carry):
        idx_v[pl.ds(i * 16, 16)] = safe
        return carry

    lax.fori_loop(0, _LH // 16, prefill, 0)

    def compact(i, off):
        mi = mask_v[pl.ds(i * 16, 16)]
        keep = mi != 0
        one = keep.astype(jnp.int32)
        csum = plsc.cumsum(one)
        pos = csum + (off - 1)
        rowids = lax.iota(jnp.int32, 16) + (start + i * 16)
        plsc.store_scatter(idx_v, [pos], rowids, mask=keep)
        return off + jnp.sum(one)

    count = lax.fori_loop(0, _LH // 16, compact, jnp.int32(0))

    nwin = (count + _W - 1) // _W

    zero = jnp.zeros((16,), jnp.float32)
    acc_v[pl.ds(0, 16)] = zero
    acc_v[pl.ds(16, 16)] = zero
    acc_v[pl.ds(32, 16)] = zero

    slots = ((rows_p0, rows_t0, sem_p0, sem_t0),
             (rows_p1, rows_t1, sem_p1, sem_t1))

    def issue(w, slot):
        rp, rt, sp, st = slot
        idx_slice = idx_v.at[pl.ds(w * _W, _W)]
        pltpu.async_copy(pred_hbm.at[idx_slice], rp, sp)
        pltpu.async_copy(target_hbm.at[idx_slice], rt, st)

    @pl.when(nwin > 0)
    def _():
        issue(0, slots[0])

    for w in range(_NWIN):
        rp, rt, sp, st = slots[w % 2]

        if w + 1 < _NWIN:
            @pl.when(w + 1 < nwin)
            def _(w=w):
                issue(w + 1, slots[(w + 1) % 2])

        @pl.when(w < nwin)
        def _(w=w, rp=rp, rt=rt, sp=sp, st=st):
            idx_slice = idx_v.at[pl.ds(w * _W, _W)]
            pltpu.make_async_copy(pred_hbm.at[idx_slice], rp, sp).wait()
            pltpu.make_async_copy(target_hbm.at[idx_slice], rt, st).wait()
            nrows = jnp.minimum(count - w * _W, _W)

            def row(r, a):
                a1, a2, a3 = a
                for ch in range(_D // 16):
                    pch = rp[r, pl.ds(ch * 16, 16)]
                    tch = rt[r, pl.ds(ch * 16, 16)]
                    a1 = a1 + pch * tch
                    a2 = a2 + pch * pch
                    a3 = a3 + tch * tch
                return (a1, a2, a3)

            a0 = (acc_v[pl.ds(0, 16)], acc_v[pl.ds(16, 16)], acc_v[pl.ds(32, 16)])
            a1, a2, a3 = lax.fori_loop(0, nrows, row, a0)
            acc_v[pl.ds(0, 16)] = a1
            acc_v[pl.ds(16, 16)] = a2
            acc_v[pl.ds(32, 16)] = a3

    acc_v[pl.ds(48, 16)] = jnp.full((16,), 1.0, jnp.float32) * count.astype(jnp.float32)
    pltpu.sync_copy(acc_v, out_hbm.at[wid])


def _sc_call(pred_flat, target_flat, mask_flat):
    mesh = plsc.VectorSubcoreMesh(core_axis_name="c", subcore_axis_name="s")
    kern = pl.kernel(
        _sc_body,
        out_type=jax.ShapeDtypeStruct((_NSUB, 64), jnp.float32),
        mesh=mesh,
        scratch_types=[
            pltpu.VMEM((_LH,), jnp.int32),
            pltpu.VMEM((_LH,), jnp.int32),
            pltpu.VMEM((_W, _D), jnp.float32),
            pltpu.VMEM((_W, _D), jnp.float32),
            pltpu.VMEM((_W, _D), jnp.float32),
            pltpu.VMEM((_W, _D), jnp.float32),
            pltpu.VMEM((64,), jnp.float32),
            pltpu.SemaphoreType.DMA,
            pltpu.SemaphoreType.DMA,
            pltpu.SemaphoreType.DMA,
            pltpu.SemaphoreType.DMA,
        ],
        compiler_params=pltpu.CompilerParams(needs_layout_passes=False),
    )
    return kern(pred_flat, target_flat, mask_flat)


def _tc_body(mask_ref, pred_ref, target_ref, out_ref, acc_ref, cnt_ref):
    b = pl.program_id(0)
    i = pl.program_id(1)

    @pl.when(jnp.logical_and(b == 0, i == 0))
    def _():
        out_ref[0, 0] = 0.0

    @pl.when(i == 0)
    def _():
        acc_ref[...] = jnp.zeros_like(acc_ref)
        cnt_ref[0] = 0.0

    m = (mask_ref[0, 0, :] != 0).astype(jnp.float32)
    mf = m[:, None]
    p = pred_ref[0]
    t = target_ref[0]
    mp = (p * mf).reshape(_BLK // 8, 8, _D)
    mt = (t * mf).reshape(_BLK // 8, 8, _D)
    pr = p.reshape(_BLK // 8, 8, _D)
    tr = t.reshape(_BLK // 8, 8, _D)
    acc_ref[0] += jnp.sum(mp * tr, axis=0)
    acc_ref[1] += jnp.sum(mp * pr, axis=0)
    acc_ref[2] += jnp.sum(mt * tr, axis=0)
    cnt_ref[0] += jnp.sum(m)

    @pl.when(i == _NB - 1)
    def _():
        dot = jnp.sum(acc_ref[0])
        pp = jnp.sum(acc_ref[1])
        tt = jnp.sum(acc_ref[2])
        cnt = cnt_ref[0]
        denom = jnp.sqrt(pp) * jnp.sqrt(tt)
        safe = jnp.where(denom > 0.0, denom, 1.0)
        loss = jnp.where(cnt > 0.0, -dot / safe, 0.0)
        out_ref[0, 0] += loss / _BS


def _tc_call(mask3, pred, target, b0, nb):
    return pl.pallas_call(
        _tc_body,
        grid=(nb, _NB),
        in_specs=[
            pl.BlockSpec((1, 1, _BLK), lambda b, i: ((b0 + b) * _NB + i, 0, 0)),
            pl.BlockSpec((1, _BLK, _D), lambda b, i: (b0 + b, i, 0)),
            pl.BlockSpec((1, _BLK, _D), lambda b, i: (b0 + b, i, 0)),
        ],
        out_specs=pl.BlockSpec(memory_space=pltpu.SMEM),
        out_shape=jax.ShapeDtypeStruct((1, 1), jnp.float32),
        scratch_shapes=[pltpu.VMEM((3, 8, _D), jnp.float32),
                        pltpu.SMEM((1,), jnp.float32)],
    )(mask3, pred, target)


def kernel(pred, target, mask):
    pred_flat = pred.reshape(_BS * _N, _D)
    target_flat = target.reshape(_BS * _N, _D)
    mask_flat = mask.reshape(_BS * _N)
    mask3 = mask.reshape(_BS * _NB, 1, _BLK)

    tc = _tc_call(mask3, pred, target, 0, _NB_TC)
    sc = _sc_call(pred_flat, target_flat, mask_flat)

    scr = sc.reshape(_NB_SC, _SPB, 4, 16)
    dot = jnp.sum(scr[:, :, 0, :], axis=(1, 2))
    pp = jnp.sum(scr[:, :, 1, :], axis=(1, 2))
    tt = jnp.sum(scr[:, :, 2, :], axis=(1, 2))
    cnt = jnp.sum(scr[:, :, 3, 0], axis=1)

    denom = jnp.sqrt(pp) * jnp.sqrt(tt)
    safe = jnp.where(denom > 0.0, denom, 1.0)
    losses = jnp.where(cnt > 0.0, -dot / safe, 0.0)
    return tc[0, 0] + jnp.sum(losses) / _BS
